# baseline (speedup 1.0000x reference)
import jax
import jax.numpy as jnp
from jax import lax
from jax.experimental import pallas as pl
from jax.experimental.pallas import tpu as pltpu
from jax.experimental.pallas import tpu_sc as plsc

_B = 16384
_D = 64
_V = 1000000
_NW = 32
_VPW = _V // _NW
_NTC = 246
_LASTTC = 7811
_TAIL0 = 999936
_C = 640
_EPOCHS = 26
_IOTA = None


def _full(s):
    return jnp.full((16,), s, dtype=jnp.int32)


def _scalar(vec, lane0):
    return jnp.sum(jnp.where(lane0, vec, 0))


def _sc_body(idx_hbm, pidx_hbm, emb_hbm, pos_hbm, tail_hbm, out_hbm,
             ibuf, vstage, pstage, vsorted, psorted, counts, offs, offs2,
             chunk_a, chunk_b, shared, posv, staging, totals,
             sem_a, sem_b, sem_s):
    wid = lax.axis_index("s") * 2 + lax.axis_index("c")
    lo = wid * _VPW
    hi = lo + _VPW
    tc_start = lax.shift_right_logical(lo, 7)
    iota = lax.iota(jnp.int32, 16)
    lane0 = iota == 0
    ones = jnp.ones((16,), jnp.int32)
    zeros = jnp.zeros((16,), jnp.int32)

    pltpu.sync_copy(pos_hbm, shared)
    pltpu.sync_copy(pidx_hbm, vstage.at[pl.ds(0, 16)])
    p_spl = vstage[pl.ds(0, 16)]
    for k in range(4):
        posv[pl.ds(k * 16, 16)] = plsc.load_gather(
            shared, [k * 16 + iota, p_spl])
    pltpu.sync_copy(tail_hbm, shared)

    def _prefill(s, _):
        for k in range(4):
            staging[s, pl.ds(_D + k * 16, 16)] = posv[pl.ds(k * 16, 16)]
        return 0
    lax.fori_loop(0, _C, _prefill, 0)

    def epoch(e_skip, is_first):
        def stage_body(st, carry):
            pltpu.sync_copy(idx_hbm.at[pl.ds(st * 2048, 2048)], ibuf)

            def group(g, carry2):
                cursor, seen = carry2
                v = ibuf[pl.ds(g * 16, 16)]
                m = (v >= lo) & (v < hi)
                cnt = jnp.sum(m.astype(jnp.int32))
                fast = (seen >= e_skip) & (seen + cnt <= e_skip + _C)

                def m_fast(_):
                    return m, cnt

                def m_slow(_):
                    pre = plsc.cumsum(m.astype(jnp.int32))
                    rank = seen + pre - 1
                    mt = m & (rank >= e_skip) & (rank < e_skip + _C)
                    return mt, jnp.sum(mt.astype(jnp.int32))

                m_take, take = lax.cond(fast, m_fast, m_slow, 0)
                plsc.store_compressed(vstage.at[pl.ds(cursor, 16)], v, mask=m_take)
                gpos = st * 2048 + g * 16 + iota
                plsc.store_compressed(pstage.at[pl.ds(cursor, 16)], gpos,
                                      mask=m_take)
                return cursor + take, seen + cnt

            return lax.fori_loop(0, 128, group, carry)

        n_e, seen_all = lax.fori_loop(
            0, 8, stage_body, (jnp.int32(0), jnp.int32(0)))
        if is_first:
            totals[0] = seen_all

        @pl.when(n_e > 0)
        def _():
            npad = lax.shift_left(
                lax.shift_right_logical(n_e + 127, 7), 7)
            nl = _full(n_e - 1)
            v_last = plsc.load_gather(vstage, [nl])
            p_last = plsc.load_gather(pstage, [nl])
            for g in range(8):
                pad_i = n_e + g * 16 + iota
                pm = pad_i < npad
                plsc.store_scatter(vstage, [pad_i], v_last, mask=pm)
                plsc.store_scatter(pstage, [pad_i], p_last, mask=pm)

            for q in range(_NTC // 16 + 1):
                counts[pl.ds(q * 16, 16)] = zeros

            def hist(q, _):
                vv = vstage[pl.ds(q * 16, 16)]
                b = lax.shift_right_logical(vv, 7) - tc_start
                plsc.addupdate_scatter(counts, [b], ones)
                return 0
            lax.fori_loop(0, lax.shift_right_logical(npad, 4), hist, 0)

            carry = 0
            for q in range(16):
                c16 = counts[pl.ds(q * 16, 16)]
                cs = plsc.cumsum(c16)
                excl = cs - c16 + carry
                offs[pl.ds(q * 16, 16)] = excl
                offs2[pl.ds(q * 16, 16)] = excl
                carry = carry + jnp.sum(c16)

            def place(s, _):
                sv = _full(s)
                v_s = plsc.load_gather(vstage, [sv])
                p_s = plsc.load_gather(pstage, [sv])
                b = lax.shift_right_logical(v_s, 7) - tc_start
                cur = plsc.load_gather(offs, [b])
                plsc.store_scatter(offs, [b], cur + 1, mask=lane0)
                plsc.store_scatter(vsorted, [cur], v_s, mask=lane0)
                plsc.store_scatter(
                    psorted,
                    [lax.shift_right_logical(cur, 7),
                     jnp.bitwise_and(cur, 127)],
                    p_s, mask=lane0)
                return 0
            lax.fori_loop(0, npad, place, 0)

            def bucket_meta(t):
                tv = _full(t)
                start = _scalar(plsc.load_gather(offs2, [tv]), lane0)
                cnt = _scalar(plsc.load_gather(counts, [tv]), lane0)
                return start, cnt

            def extract(buf, t, base_col):
                start, cnt = bucket_meta(t)

                def elem(s, _):
                    sv = _full(s)
                    v_s = plsc.load_gather(vsorted, [sv])
                    c = v_s - base_col
                    for k in range(4):
                        staging[s, pl.ds(k * 16, 16)] = plsc.load_gather(
                            buf, [k * 16 + iota, c])
                    return 0
                lax.fori_loop(start, start + cnt, elem, 0)

            def issue(t, buf, sem):
                tc = jnp.minimum(tc_start + t, _LASTTC)
                off = pl.multiple_of(tc * 128, 128)
                pltpu.async_copy(emb_hbm.at[:, pl.ds(off, 128)], buf, sem)

            def drain(buf, sem):
                pltpu.make_async_copy(
                    emb_hbm.at[:, pl.ds(0, 128)], buf, sem).wait()

            def process(t, buf):
                @pl.when(tc_start + t <= _LASTTC)
                def _():
                    extract(buf, t, (tc_start + t) * 128)

            issue(0, chunk_a, sem_a)
            issue(1, chunk_b, sem_b)

            def pair(i, _):
                t0 = 2 * i
                drain(chunk_a, sem_a)
                process(t0, chunk_a)

                @pl.when(t0 + 2 < _NTC)
                def _():
                    issue(t0 + 2, chunk_a, sem_a)
                drain(chunk_b, sem_b)
                process(t0 + 1, chunk_b)

                @pl.when(t0 + 3 < _NTC)
                def _():
                    issue(t0 + 3, chunk_b, sem_b)
                return 0
            lax.fori_loop(0, _NTC // 2, pair, 0)

            b_tail = 7812 - tc_start

            @pl.when((b_tail >= 0) & (b_tail < _NTC))
            def _():
                start, cnt = bucket_meta(b_tail)

                def elem(s, _):
                    sv = _full(s)
                    v_s = plsc.load_gather(vsorted, [sv])
                    c = v_s - _TAIL0
                    for k in range(4):
                        staging[s, pl.ds(k * 16, 16)] = plsc.load_gather(
                            shared, [k * 16 + iota, c])
                    return 0
                lax.fori_loop(start, start + cnt, elem, 0)

            def flush(q, _):
                pltpu.async_copy(staging.at[pl.ds(q * 128, 128)],
                                 out_hbm.at[psorted.at[q]], sem_s).wait()
                return 0
            lax.fori_loop(0, lax.shift_right_logical(npad, 7), flush, 0)

    epoch(0, True)

    def later(e, _):
        @pl.when(totals[0] > e * _C)
        def _():
            epoch(e * _C, False)
        return 0
    lax.fori_loop(1, _EPOCHS, later, 0)


def kernel(input, input_position, input_embedding, positional_embedding):
    idx = input.astype(jnp.int32)
    pidx = jnp.full((16,), input_position, dtype=jnp.int32)
    emb_t = input_embedding.T
    pos128 = jnp.pad(positional_embedding.T, ((0, 0), (0, 28)))
    tail128 = jnp.pad(input_embedding[_TAIL0:].T, ((0, 0), (0, 64)))
    mesh = plsc.VectorSubcoreMesh(core_axis_name="c", subcore_axis_name="s")
    f = pl.kernel(
        _sc_body,
        out_type=jax.ShapeDtypeStruct((_B, 2 * _D), jnp.float32),
        mesh=mesh,
        compiler_params=pltpu.CompilerParams(use_tc_tiling_on_sc=True,
                                             needs_layout_passes=False),
        scratch_types=[
            pltpu.VMEM((2048,), jnp.int32),
            pltpu.VMEM((_C + 144,), jnp.int32),
            pltpu.VMEM((_C + 144,), jnp.int32),
            pltpu.VMEM((_C,), jnp.int32),
            pltpu.VMEM((5, 128), jnp.int32),
            pltpu.VMEM((256,), jnp.int32),
            pltpu.VMEM((256,), jnp.int32),
            pltpu.VMEM((256,), jnp.int32),
            pltpu.VMEM((_D, 128), jnp.float32),
            pltpu.VMEM((_D, 128), jnp.float32),
            pltpu.VMEM((_D, 128), jnp.float32),
            pltpu.VMEM((_D,), jnp.float32),
            pltpu.VMEM((_C, 2 * _D), jnp.float32),
            pltpu.SMEM((1,), jnp.int32),
            pltpu.SemaphoreType.DMA,
            pltpu.SemaphoreType.DMA,
            pltpu.SemaphoreType.DMA,
        ],
    )
    return f(idx, pidx, emb_t, pos128, tail128)


# speedup vs baseline: 1.0519x; 1.0519x over previous
import jax
import jax.numpy as jnp
from jax import lax
from jax.experimental import pallas as pl
from jax.experimental.pallas import tpu as pltpu
from jax.experimental.pallas import tpu_sc as plsc

_B = 16384
_D = 64
_V = 1000000
_NW = 32
_VPW = _V // _NW
_NTC = 246
_LASTTC = 7811
_TAIL0 = 999936
_C = 640
_EPOCHS = 26
_IOTA = None


def _full(s):
    return jnp.full((16,), s, dtype=jnp.int32)


def _scalar(vec, lane0):
    return jnp.sum(jnp.where(lane0, vec, 0))


def _sc_body(idx_hbm, pidx_hbm, emb_hbm, pos_hbm, tail_hbm, out_hbm,
             ibuf, vstage, pstage, vsorted, psorted, counts, offs, offs2,
             chunk_a, chunk_b, shared, posv, staging, totals,
             sem_a, sem_b, sem_s):
    wid = lax.axis_index("s") * 2 + lax.axis_index("c")
    lo = wid * _VPW
    hi = lo + _VPW
    tc_start = lax.shift_right_logical(lo, 7)
    iota = lax.iota(jnp.int32, 16)
    lane0 = iota == 0
    ones = jnp.ones((16,), jnp.int32)
    zeros = jnp.zeros((16,), jnp.int32)

    pltpu.sync_copy(pos_hbm, shared)
    pltpu.sync_copy(pidx_hbm, vstage.at[pl.ds(0, 16)])
    p_spl = vstage[pl.ds(0, 16)]
    for k in range(4):
        posv[pl.ds(k * 16, 16)] = plsc.load_gather(
            shared, [k * 16 + iota, p_spl])
    pltpu.sync_copy(tail_hbm, shared)

    def _prefill(i, _):
        for u in range(8):
            for k in range(4):
                staging[i * 8 + u, pl.ds(_D + k * 16, 16)] = (
                    posv[pl.ds(k * 16, 16)])
        return 0
    lax.fori_loop(0, _C // 8, _prefill, 0)

    def epoch(e_skip, is_first):
        def stage_body(st, carry):
            pltpu.sync_copy(idx_hbm.at[pl.ds(st * 2048, 2048)], ibuf)

            def group(g8, carry2):
                cursor, seen = carry2
                for u in range(8):
                    g = g8 * 8 + u
                    v = ibuf[pl.ds(g * 16, 16)]
                    m = (v >= lo) & (v < hi)
                    cnt = jnp.sum(m.astype(jnp.int32))
                    fast = (seen >= e_skip) & (seen + cnt <= e_skip + _C)

                    def m_fast(_, m=m, cnt=cnt):
                        return m, cnt

                    def m_slow(_, m=m, seen=seen):
                        pre = plsc.cumsum(m.astype(jnp.int32))
                        rank = seen + pre - 1
                        mt = m & (rank >= e_skip) & (rank < e_skip + _C)
                        return mt, jnp.sum(mt.astype(jnp.int32))

                    m_take, take = lax.cond(fast, m_fast, m_slow, 0)
                    plsc.store_compressed(vstage.at[pl.ds(cursor, 16)], v,
                                          mask=m_take)
                    gpos = st * 2048 + g * 16 + iota
                    plsc.store_compressed(pstage.at[pl.ds(cursor, 16)], gpos,
                                          mask=m_take)
                    cursor = cursor + take
                    seen = seen + cnt
                return cursor, seen

            return lax.fori_loop(0, 16, group, carry)

        n_e, seen_all = lax.fori_loop(
            0, 8, stage_body, (jnp.int32(0), jnp.int32(0)))
        if is_first:
            totals[0] = seen_all

        @pl.when(n_e > 0)
        def _():
            npad = lax.shift_left(
                lax.shift_right_logical(n_e + 127, 7), 7)
            nl = _full(n_e - 1)
            v_last = plsc.load_gather(vstage, [nl])
            p_last = plsc.load_gather(pstage, [nl])
            for g in range(8):
                pad_i = n_e + g * 16 + iota
                pm = pad_i < npad
                plsc.store_scatter(vstage, [pad_i], v_last, mask=pm)
                plsc.store_scatter(pstage, [pad_i], p_last, mask=pm)

            for q in range(_NTC // 16 + 1):
                counts[pl.ds(q * 16, 16)] = zeros

            def hist(q8, _):
                for u in range(8):
                    vv = vstage[pl.ds((q8 * 8 + u) * 16, 16)]
                    b = lax.shift_right_logical(vv, 7) - tc_start
                    plsc.addupdate_scatter(counts, [b], ones)
                return 0
            lax.fori_loop(0, lax.shift_right_logical(npad, 7), hist, 0)

            carry = 0
            for q in range(16):
                c16 = counts[pl.ds(q * 16, 16)]
                cs = plsc.cumsum(c16)
                excl = cs - c16 + carry
                offs[pl.ds(q * 16, 16)] = excl
                offs2[pl.ds(q * 16, 16)] = excl
                carry = carry + jnp.sum(c16)

            def place(i4, _):
                for u in range(4):
                    sv = _full(i4 * 4 + u)
                    v_s = plsc.load_gather(vstage, [sv])
                    p_s = plsc.load_gather(pstage, [sv])
                    b = lax.shift_right_logical(v_s, 7) - tc_start
                    cur = plsc.load_gather(offs, [b])
                    plsc.store_scatter(offs, [b], cur + 1, mask=lane0)
                    plsc.store_scatter(vsorted, [cur], v_s, mask=lane0)
                    plsc.store_scatter(
                        psorted,
                        [lax.shift_right_logical(cur, 7),
                         jnp.bitwise_and(cur, 127)],
                        p_s, mask=lane0)
                return 0
            lax.fori_loop(0, lax.shift_right_logical(npad, 2), place, 0)

            def bucket_meta(t):
                tv = _full(t)
                start = _scalar(plsc.load_gather(offs2, [tv]), lane0)
                cnt = _scalar(plsc.load_gather(counts, [tv]), lane0)
                return start, cnt

            def extract(buf, t, base_col):
                start, cnt = bucket_meta(t)

                def elem(s, _):
                    sv = _full(s)
                    v_s = plsc.load_gather(vsorted, [sv])
                    c = v_s - base_col
                    for k in range(4):
                        staging[s, pl.ds(k * 16, 16)] = plsc.load_gather(
                            buf, [k * 16 + iota, c])
                    return 0
                lax.fori_loop(start, start + cnt, elem, 0)

            def issue(t, buf, sem):
                tc = jnp.minimum(tc_start + t, _LASTTC)
                off = pl.multiple_of(tc * 128, 128)
                pltpu.async_copy(emb_hbm.at[:, pl.ds(off, 128)], buf, sem)

            def drain(buf, sem):
                pltpu.make_async_copy(
                    emb_hbm.at[:, pl.ds(0, 128)], buf, sem).wait()

            def process(t, buf):
                @pl.when(tc_start + t <= _LASTTC)
                def _():
                    extract(buf, t, (tc_start + t) * 128)

            issue(0, chunk_a, sem_a)
            issue(1, chunk_b, sem_b)

            def pair(i, _):
                t0 = 2 * i
                drain(chunk_a, sem_a)
                process(t0, chunk_a)

                @pl.when(t0 + 2 < _NTC)
                def _():
                    issue(t0 + 2, chunk_a, sem_a)
                drain(chunk_b, sem_b)
                process(t0 + 1, chunk_b)

                @pl.when(t0 + 3 < _NTC)
                def _():
                    issue(t0 + 3, chunk_b, sem_b)
                return 0
            lax.fori_loop(0, _NTC // 2, pair, 0)

            b_tail = 7812 - tc_start

            @pl.when((b_tail >= 0) & (b_tail < _NTC))
            def _():
                start, cnt = bucket_meta(b_tail)

                def elem(s, _):
                    sv = _full(s)
                    v_s = plsc.load_gather(vsorted, [sv])
                    c = v_s - _TAIL0
                    for k in range(4):
                        staging[s, pl.ds(k * 16, 16)] = plsc.load_gather(
                            shared, [k * 16 + iota, c])
                    return 0
                lax.fori_loop(start, start + cnt, elem, 0)

            def flush(q, _):
                pltpu.async_copy(staging.at[pl.ds(q * 128, 128)],
                                 out_hbm.at[psorted.at[q]], sem_s).wait()
                return 0
            lax.fori_loop(0, lax.shift_right_logical(npad, 7), flush, 0)

    epoch(0, True)

    def later(e, _):
        @pl.when(totals[0] > e * _C)
        def _():
            epoch(e * _C, False)
        return 0
    lax.fori_loop(1, _EPOCHS, later, 0)


def kernel(input, input_position, input_embedding, positional_embedding):
    idx = input.astype(jnp.int32)
    pidx = jnp.full((16,), input_position, dtype=jnp.int32)
    emb_t = input_embedding.T
    pos128 = jnp.pad(positional_embedding.T, ((0, 0), (0, 28)))
    tail128 = jnp.pad(input_embedding[_TAIL0:].T, ((0, 0), (0, 64)))
    mesh = plsc.VectorSubcoreMesh(core_axis_name="c", subcore_axis_name="s")
    f = pl.kernel(
        _sc_body,
        out_type=jax.ShapeDtypeStruct((_B, 2 * _D), jnp.float32),
        mesh=mesh,
        compiler_params=pltpu.CompilerParams(use_tc_tiling_on_sc=True,
                                             needs_layout_passes=False),
        scratch_types=[
            pltpu.VMEM((2048,), jnp.int32),
            pltpu.VMEM((_C + 144,), jnp.int32),
            pltpu.VMEM((_C + 144,), jnp.int32),
            pltpu.VMEM((_C,), jnp.int32),
            pltpu.VMEM((5, 128), jnp.int32),
            pltpu.VMEM((256,), jnp.int32),
            pltpu.VMEM((256,), jnp.int32),
            pltpu.VMEM((256,), jnp.int32),
            pltpu.VMEM((_D, 128), jnp.float32),
            pltpu.VMEM((_D, 128), jnp.float32),
            pltpu.VMEM((_D, 128), jnp.float32),
            pltpu.VMEM((_D,), jnp.float32),
            pltpu.VMEM((_C, 2 * _D), jnp.float32),
            pltpu.SMEM((1,), jnp.int32),
            pltpu.SemaphoreType.DMA,
            pltpu.SemaphoreType.DMA,
            pltpu.SemaphoreType.DMA,
        ],
    )
    return f(idx, pidx, emb_t, pos128, tail128)


# speedup vs baseline: 1.3298x; 1.2642x over previous
import jax
import jax.numpy as jnp
from jax import lax
from jax.experimental import pallas as pl
from jax.experimental.pallas import tpu as pltpu
from jax.experimental.pallas import tpu_sc as plsc

_B = 16384
_D = 64
_V = 1000000
_NW = 32
_VPW = _V // _NW
_NTC = 246
_LASTTC = 7811
_TAIL0 = 999936
_C = 640
_EPOCHS = 26
_IOTA = None


def _full(s):
    return jnp.full((16,), s, dtype=jnp.int32)


def _scalar(vec, lane0):
    return jnp.sum(jnp.where(lane0, vec, 0))


def _sc_body(idx_hbm, pidx_hbm, emb_hbm, pos_hbm, tail_hbm, out_hbm,
             ibuf, vstage, pstage, vsorted, psorted, counts, offs, offs2,
             chunk_0, chunk_1, chunk_2, chunk_3, shared, posv, staging, totals,
             sem_0, sem_1, sem_2, sem_3, sem_s):
    wid = lax.axis_index("s") * 2 + lax.axis_index("c")
    lo = wid * _VPW
    hi = lo + _VPW
    tc_start = lax.shift_right_logical(lo, 7)
    iota = lax.iota(jnp.int32, 16)
    lane0 = iota == 0
    ones = jnp.ones((16,), jnp.int32)
    zeros = jnp.zeros((16,), jnp.int32)

    pltpu.sync_copy(pos_hbm, shared)
    pltpu.sync_copy(pidx_hbm, vstage.at[pl.ds(0, 16)])
    p_spl = vstage[pl.ds(0, 16)]
    for k in range(4):
        posv[pl.ds(k * 16, 16)] = plsc.load_gather(
            shared, [k * 16 + iota, p_spl])
    pltpu.sync_copy(tail_hbm, shared)

    def _prefill(i, _):
        for u in range(8):
            for k in range(4):
                staging[i * 8 + u, pl.ds(_D + k * 16, 16)] = (
                    posv[pl.ds(k * 16, 16)])
        return 0
    lax.fori_loop(0, _C // 8, _prefill, 0)

    def epoch(e_skip, is_first):
        def stage_body(st, carry):
            pltpu.sync_copy(idx_hbm.at[pl.ds(st * 2048, 2048)], ibuf)

            def group(g8, carry2):
                cursor, seen = carry2
                for u in range(8):
                    g = g8 * 8 + u
                    v = ibuf[pl.ds(g * 16, 16)]
                    m = (v >= lo) & (v < hi)
                    cnt = jnp.sum(m.astype(jnp.int32))
                    fast = (seen >= e_skip) & (seen + cnt <= e_skip + _C)

                    def m_fast(_, m=m, cnt=cnt):
                        return m, cnt

                    def m_slow(_, m=m, seen=seen):
                        pre = plsc.cumsum(m.astype(jnp.int32))
                        rank = seen + pre - 1
                        mt = m & (rank >= e_skip) & (rank < e_skip + _C)
                        return mt, jnp.sum(mt.astype(jnp.int32))

                    m_take, take = lax.cond(fast, m_fast, m_slow, 0)
                    plsc.store_compressed(vstage.at[pl.ds(cursor, 16)], v,
                                          mask=m_take)
                    gpos = st * 2048 + g * 16 + iota
                    plsc.store_compressed(pstage.at[pl.ds(cursor, 16)], gpos,
                                          mask=m_take)
                    cursor = cursor + take
                    seen = seen + cnt
                return cursor, seen

            return lax.fori_loop(0, 16, group, carry)

        n_e, seen_all = lax.fori_loop(
            0, 8, stage_body, (jnp.int32(0), jnp.int32(0)))
        if is_first:
            totals[0] = seen_all

        @pl.when(n_e > 0)
        def _():
            npad = lax.shift_left(
                lax.shift_right_logical(n_e + 127, 7), 7)
            nl = _full(n_e - 1)
            v_last = plsc.load_gather(vstage, [nl])
            p_last = plsc.load_gather(pstage, [nl])
            for g in range(8):
                pad_i = n_e + g * 16 + iota
                pm = pad_i < npad
                plsc.store_scatter(vstage, [pad_i], v_last, mask=pm)
                plsc.store_scatter(pstage, [pad_i], p_last, mask=pm)

            for q in range(_NTC // 16 + 1):
                counts[pl.ds(q * 16, 16)] = zeros

            def hist(q8, _):
                for u in range(8):
                    vv = vstage[pl.ds((q8 * 8 + u) * 16, 16)]
                    b = lax.shift_right_logical(vv, 7) - tc_start
                    plsc.addupdate_scatter(counts, [b], ones)
                return 0
            lax.fori_loop(0, lax.shift_right_logical(npad, 7), hist, 0)

            carry = 0
            for q in range(16):
                c16 = counts[pl.ds(q * 16, 16)]
                cs = plsc.cumsum(c16)
                excl = cs - c16 + carry
                offs[pl.ds(q * 16, 16)] = excl
                offs2[pl.ds(q * 16, 16)] = excl
                carry = carry + jnp.sum(c16)

            def place(i4, _):
                for u in range(4):
                    sv = _full(i4 * 4 + u)
                    v_s = plsc.load_gather(vstage, [sv])
                    p_s = plsc.load_gather(pstage, [sv])
                    b = lax.shift_right_logical(v_s, 7) - tc_start
                    cur = plsc.load_gather(offs, [b])
                    plsc.store_scatter(offs, [b], cur + 1, mask=lane0)
                    plsc.store_scatter(vsorted, [cur], v_s, mask=lane0)
                    plsc.store_scatter(
                        psorted,
                        [lax.shift_right_logical(cur, 7),
                         jnp.bitwise_and(cur, 127)],
                        p_s, mask=lane0)
                return 0
            lax.fori_loop(0, lax.shift_right_logical(npad, 2), place, 0)

            def bucket_meta(t):
                tv = _full(t)
                start = _scalar(plsc.load_gather(offs2, [tv]), lane0)
                cnt = _scalar(plsc.load_gather(counts, [tv]), lane0)
                return start, cnt

            def extract(buf, t, base_col):
                start, cnt = bucket_meta(t)

                def elem(s, _):
                    sv = _full(s)
                    v_s = plsc.load_gather(vsorted, [sv])
                    c = v_s - base_col
                    for k in range(4):
                        staging[s, pl.ds(k * 16, 16)] = plsc.load_gather(
                            buf, [k * 16 + iota, c])
                    return 0
                lax.fori_loop(start, start + cnt, elem, 0)

            def issue(t, buf, sem):
                tc = jnp.minimum(tc_start + t, _LASTTC)
                off = pl.multiple_of(tc * 128, 128)
                pltpu.async_copy(emb_hbm.at[:, pl.ds(off, 128)], buf, sem)

            def drain(buf, sem):
                pltpu.make_async_copy(
                    emb_hbm.at[:, pl.ds(0, 128)], buf, sem).wait()

            def process(t, buf):
                @pl.when(tc_start + t <= _LASTTC)
                def _():
                    extract(buf, t, (tc_start + t) * 128)

            ring = [(chunk_0, sem_0), (chunk_1, sem_1),
                    (chunk_2, sem_2), (chunk_3, sem_3)]
            for u in range(4):
                issue(u, ring[u][0], ring[u][1])

            def quad(i, _):
                for u in range(4):
                    t = 4 * i + u
                    buf, sem = ring[u]

                    @pl.when(t < _NTC)
                    def _(t=t, buf=buf, sem=sem):
                        drain(buf, sem)
                        process(t, buf)

                        @pl.when(t + 4 < _NTC)
                        def _():
                            issue(t + 4, buf, sem)
                return 0
            lax.fori_loop(0, (_NTC + 3) // 4, quad, 0)

            b_tail = 7812 - tc_start

            @pl.when((b_tail >= 0) & (b_tail < _NTC))
            def _():
                start, cnt = bucket_meta(b_tail)

                def elem(s, _):
                    sv = _full(s)
                    v_s = plsc.load_gather(vsorted, [sv])
                    c = v_s - _TAIL0
                    for k in range(4):
                        staging[s, pl.ds(k * 16, 16)] = plsc.load_gather(
                            shared, [k * 16 + iota, c])
                    return 0
                lax.fori_loop(start, start + cnt, elem, 0)

            def flush(q, _):
                pltpu.async_copy(staging.at[pl.ds(q * 128, 128)],
                                 out_hbm.at[psorted.at[q]], sem_s).wait()
                return 0
            lax.fori_loop(0, lax.shift_right_logical(npad, 7), flush, 0)

    epoch(0, True)

    def later(e, _):
        @pl.when(totals[0] > e * _C)
        def _():
            epoch(e * _C, False)
        return 0
    lax.fori_loop(1, _EPOCHS, later, 0)


def kernel(input, input_position, input_embedding, positional_embedding):
    idx = input.astype(jnp.int32)
    pidx = jnp.full((16,), input_position, dtype=jnp.int32)
    emb_t = input_embedding.T
    pos128 = jnp.pad(positional_embedding.T, ((0, 0), (0, 28)))
    tail128 = jnp.pad(input_embedding[_TAIL0:].T, ((0, 0), (0, 64)))
    mesh = plsc.VectorSubcoreMesh(core_axis_name="c", subcore_axis_name="s")
    f = pl.kernel(
        _sc_body,
        out_type=jax.ShapeDtypeStruct((_B, 2 * _D), jnp.float32),
        mesh=mesh,
        compiler_params=pltpu.CompilerParams(use_tc_tiling_on_sc=True,
                                             needs_layout_passes=False),
        scratch_types=[
            pltpu.VMEM((2048,), jnp.int32),
            pltpu.VMEM((_C + 144,), jnp.int32),
            pltpu.VMEM((_C + 144,), jnp.int32),
            pltpu.VMEM((_C,), jnp.int32),
            pltpu.VMEM((5, 128), jnp.int32),
            pltpu.VMEM((256,), jnp.int32),
            pltpu.VMEM((256,), jnp.int32),
            pltpu.VMEM((256,), jnp.int32),
            pltpu.VMEM((_D, 128), jnp.float32),
            pltpu.VMEM((_D, 128), jnp.float32),
            pltpu.VMEM((_D, 128), jnp.float32),
            pltpu.VMEM((_D, 128), jnp.float32),
            pltpu.VMEM((_D, 128), jnp.float32),
            pltpu.VMEM((_D,), jnp.float32),
            pltpu.VMEM((_C, 2 * _D), jnp.float32),
            pltpu.SMEM((1,), jnp.int32),
            pltpu.SemaphoreType.DMA,
            pltpu.SemaphoreType.DMA,
            pltpu.SemaphoreType.DMA,
            pltpu.SemaphoreType.DMA,
            pltpu.SemaphoreType.DMA,
        ],
    )
    return f(idx, pidx, emb_t, pos128, tail128)


# speedup vs baseline: 1.3687x; 1.0292x over previous
import jax
import jax.numpy as jnp
from jax import lax
from jax.experimental import pallas as pl
from jax.experimental.pallas import tpu as pltpu
from jax.experimental.pallas import tpu_sc as plsc

_B = 16384
_D = 64
_V = 1000000
_NW = 32
_VPW = _V // _NW
_NTC = 246
_LASTTC = 7811
_TAIL0 = 999936
_C = 640
_EPOCHS = 26
_IOTA = None


def _full(s):
    return jnp.full((16,), s, dtype=jnp.int32)


def _scalar(vec, lane0):
    return jnp.sum(jnp.where(lane0, vec, 0))


def _sc_body(idx_hbm, pidx_hbm, emb_hbm, pos_hbm, tail_hbm, out_hbm,
             ibuf, vstage, pstage, vsorted, psorted, counts, offs, offs2,
             chunk_0, chunk_1, chunk_2, chunk_3, shared, posv, staging, totals,
             sem_0, sem_1, sem_2, sem_3, sem_s, sem_i):
    wid = lax.axis_index("s") * 2 + lax.axis_index("c")
    lo = wid * _VPW
    hi = lo + _VPW
    tc_start = lax.shift_right_logical(lo, 7)
    iota = lax.iota(jnp.int32, 16)
    lane0 = iota == 0
    ones = jnp.ones((16,), jnp.int32)
    zeros = jnp.zeros((16,), jnp.int32)

    pltpu.sync_copy(pos_hbm, shared)
    pltpu.sync_copy(pidx_hbm, vstage.at[pl.ds(0, 16)])
    p_spl = vstage[pl.ds(0, 16)]
    for k in range(4):
        posv[pl.ds(k * 16, 16)] = plsc.load_gather(
            shared, [k * 16 + iota, p_spl])
    pltpu.sync_copy(tail_hbm, shared)

    def _prefill(i, _):
        for u in range(8):
            for k in range(4):
                staging[i * 8 + u, pl.ds(_D + k * 16, 16)] = (
                    posv[pl.ds(k * 16, 16)])
        return 0
    lax.fori_loop(0, _C // 8, _prefill, 0)

    def epoch(e_skip, is_first):
        pltpu.async_copy(idx_hbm.at[pl.ds(0, 1024)], ibuf.at[0], sem_i)

        def stage_body(st, carry):
            par = jnp.bitwise_and(st, 1)
            pltpu.make_async_copy(
                idx_hbm.at[pl.ds(0, 1024)], ibuf.at[par], sem_i).wait()

            @pl.when(st + 1 < 16)
            def _():
                pltpu.async_copy(idx_hbm.at[pl.ds((st + 1) * 1024, 1024)],
                                 ibuf.at[1 - par], sem_i)

            def group(g8, carry2):
                cursor, seen = carry2
                for u in range(8):
                    g = g8 * 8 + u
                    v = ibuf[par, pl.ds(g * 16, 16)]
                    m = (v >= lo) & (v < hi)
                    cnt = jnp.sum(m.astype(jnp.int32))
                    fast = (seen >= e_skip) & (seen + cnt <= e_skip + _C)

                    def m_fast(_, m=m, cnt=cnt):
                        return m, cnt

                    def m_slow(_, m=m, seen=seen):
                        pre = plsc.cumsum(m.astype(jnp.int32))
                        rank = seen + pre - 1
                        mt = m & (rank >= e_skip) & (rank < e_skip + _C)
                        return mt, jnp.sum(mt.astype(jnp.int32))

                    m_take, take = lax.cond(fast, m_fast, m_slow, 0)
                    plsc.store_compressed(vstage.at[pl.ds(cursor, 16)], v,
                                          mask=m_take)
                    gpos = st * 1024 + g * 16 + iota
                    plsc.store_compressed(pstage.at[pl.ds(cursor, 16)], gpos,
                                          mask=m_take)
                    cursor = cursor + take
                    seen = seen + cnt
                return cursor, seen

            return lax.fori_loop(0, 8, group, carry)

        n_e, seen_all = lax.fori_loop(
            0, 16, stage_body, (jnp.int32(0), jnp.int32(0)))
        if is_first:
            totals[0] = seen_all

        @pl.when(n_e > 0)
        def _():
            npad = lax.shift_left(
                lax.shift_right_logical(n_e + 127, 7), 7)
            nl = _full(n_e - 1)
            v_last = plsc.load_gather(vstage, [nl])
            p_last = plsc.load_gather(pstage, [nl])
            for g in range(8):
                pad_i = n_e + g * 16 + iota
                pm = pad_i < npad
                plsc.store_scatter(vstage, [pad_i], v_last, mask=pm)
                plsc.store_scatter(pstage, [pad_i], p_last, mask=pm)

            for q in range(_NTC // 16 + 1):
                counts[pl.ds(q * 16, 16)] = zeros

            def hist(q8, _):
                for u in range(8):
                    vv = vstage[pl.ds((q8 * 8 + u) * 16, 16)]
                    b = lax.shift_right_logical(vv, 7) - tc_start
                    plsc.addupdate_scatter(counts, [b], ones)
                return 0
            lax.fori_loop(0, lax.shift_right_logical(npad, 7), hist, 0)

            carry = 0
            for q in range(16):
                c16 = counts[pl.ds(q * 16, 16)]
                cs = plsc.cumsum(c16)
                excl = cs - c16 + carry
                offs[pl.ds(q * 16, 16)] = excl
                offs2[pl.ds(q * 16, 16)] = excl
                carry = carry + jnp.sum(c16)

            def place(i4, _):
                for u in range(4):
                    sv = _full(i4 * 4 + u)
                    v_s = plsc.load_gather(vstage, [sv])
                    p_s = plsc.load_gather(pstage, [sv])
                    b = lax.shift_right_logical(v_s, 7) - tc_start
                    cur = plsc.load_gather(offs, [b])
                    plsc.store_scatter(offs, [b], cur + 1, mask=lane0)
                    plsc.store_scatter(vsorted, [cur], v_s, mask=lane0)
                    plsc.store_scatter(
                        psorted,
                        [lax.shift_right_logical(cur, 7),
                         jnp.bitwise_and(cur, 127)],
                        p_s, mask=lane0)
                return 0
            lax.fori_loop(0, lax.shift_right_logical(npad, 2), place, 0)

            def bucket_meta(t):
                tv = _full(t)
                start = _scalar(plsc.load_gather(offs2, [tv]), lane0)
                cnt = _scalar(plsc.load_gather(counts, [tv]), lane0)
                return start, cnt

            def extract(buf, t, base_col):
                start, cnt = bucket_meta(t)

                def elem(s, _):
                    sv = _full(s)
                    v_s = plsc.load_gather(vsorted, [sv])
                    c = v_s - base_col
                    for k in range(4):
                        staging[s, pl.ds(k * 16, 16)] = plsc.load_gather(
                            buf, [k * 16 + iota, c])
                    return 0
                lax.fori_loop(start, start + cnt, elem, 0)

            def issue(t, buf, sem):
                tc = jnp.minimum(tc_start + t, _LASTTC)
                off = pl.multiple_of(tc * 128, 128)
                pltpu.async_copy(emb_hbm.at[:, pl.ds(off, 128)], buf, sem)

            def drain(buf, sem):
                pltpu.make_async_copy(
                    emb_hbm.at[:, pl.ds(0, 128)], buf, sem).wait()

            def process(t, buf):
                @pl.when(tc_start + t <= _LASTTC)
                def _():
                    extract(buf, t, (tc_start + t) * 128)

            ring = [(chunk_0, sem_0), (chunk_1, sem_1),
                    (chunk_2, sem_2), (chunk_3, sem_3)]
            for u in range(4):
                issue(u, ring[u][0], ring[u][1])

            def quad(i, _):
                for u in range(4):
                    t = 4 * i + u
                    buf, sem = ring[u]

                    @pl.when(t < _NTC)
                    def _(t=t, buf=buf, sem=sem):
                        drain(buf, sem)
                        process(t, buf)

                        @pl.when(t + 4 < _NTC)
                        def _():
                            issue(t + 4, buf, sem)
                return 0
            lax.fori_loop(0, (_NTC + 3) // 4, quad, 0)

            b_tail = 7812 - tc_start

            @pl.when((b_tail >= 0) & (b_tail < _NTC))
            def _():
                start, cnt = bucket_meta(b_tail)

                def elem(s, _):
                    sv = _full(s)
                    v_s = plsc.load_gather(vsorted, [sv])
                    c = v_s - _TAIL0
                    for k in range(4):
                        staging[s, pl.ds(k * 16, 16)] = plsc.load_gather(
                            shared, [k * 16 + iota, c])
                    return 0
                lax.fori_loop(start, start + cnt, elem, 0)

            def flush(q, _):
                pltpu.async_copy(staging.at[pl.ds(q * 128, 128)],
                                 out_hbm.at[psorted.at[q]], sem_s).wait()
                return 0
            lax.fori_loop(0, lax.shift_right_logical(npad, 7), flush, 0)

    epoch(0, True)

    def later(e, _):
        @pl.when(totals[0] > e * _C)
        def _():
            epoch(e * _C, False)
        return 0
    lax.fori_loop(1, _EPOCHS, later, 0)


def kernel(input, input_position, input_embedding, positional_embedding):
    idx = input.astype(jnp.int32)
    pidx = jnp.full((16,), input_position, dtype=jnp.int32)
    emb_t = input_embedding.T
    pos128 = jnp.pad(positional_embedding.T, ((0, 0), (0, 28)))
    tail128 = jnp.pad(input_embedding[_TAIL0:].T, ((0, 0), (0, 64)))
    mesh = plsc.VectorSubcoreMesh(core_axis_name="c", subcore_axis_name="s")
    f = pl.kernel(
        _sc_body,
        out_type=jax.ShapeDtypeStruct((_B, 2 * _D), jnp.float32),
        mesh=mesh,
        compiler_params=pltpu.CompilerParams(use_tc_tiling_on_sc=True,
                                             needs_layout_passes=False),
        scratch_types=[
            pltpu.VMEM((2, 1024), jnp.int32),
            pltpu.VMEM((_C + 144,), jnp.int32),
            pltpu.VMEM((_C + 144,), jnp.int32),
            pltpu.VMEM((_C,), jnp.int32),
            pltpu.VMEM((5, 128), jnp.int32),
            pltpu.VMEM((256,), jnp.int32),
            pltpu.VMEM((256,), jnp.int32),
            pltpu.VMEM((256,), jnp.int32),
            pltpu.VMEM((_D, 128), jnp.float32),
            pltpu.VMEM((_D, 128), jnp.float32),
            pltpu.VMEM((_D, 128), jnp.float32),
            pltpu.VMEM((_D, 128), jnp.float32),
            pltpu.VMEM((_D, 128), jnp.float32),
            pltpu.VMEM((_D,), jnp.float32),
            pltpu.VMEM((_C, 2 * _D), jnp.float32),
            pltpu.SMEM((1,), jnp.int32),
            pltpu.SemaphoreType.DMA,
            pltpu.SemaphoreType.DMA,
            pltpu.SemaphoreType.DMA,
            pltpu.SemaphoreType.DMA,
            pltpu.SemaphoreType.DMA,
            pltpu.SemaphoreType.DMA,
        ],
    )
    return f(idx, pidx, emb_t, pos128, tail128)
